# baseline (speedup 1.0000x reference)
import functools

import jax
import jax.numpy as jnp
from jax import lax
from jax.experimental import pallas as pl
from jax.experimental.pallas import tpu as pltpu
from jax.experimental.pallas import tpu_sc as plsc

N = 10000
E = 320000
D_IN = 128
D_HID = 128
D_OUT = 64
EPS = 1e-5

NC = 2
NS = 16
NW = NC * NS
LANES = 16
CHUNK = 128

E_PAD = 327680
NROW = E_PAD // CHUNK

DEG_CH = NROW // NW

AGG_CH = NROW // NS
SLAB = 40
NSLAB = AGG_CH // SLAB
NBUF = 2

ROWS_DEG = 640
N_DEG = NS * ROWS_DEG
ROWS_AGG = 632
N_AGG = NS * ROWS_AGG

_mesh = plsc.VectorSubcoreMesh(core_axis_name="c", subcore_axis_name="s")


@functools.partial(
    pl.kernel,
    out_type=jax.ShapeDtypeStruct((NC, N_DEG), jnp.float32),
    mesh=_mesh,
    scratch_types=[
        pltpu.VMEM((DEG_CH, CHUNK), jnp.int32),
        pltpu.VMEM((CHUNK,), jnp.float32),
        pltpu.VMEM_SHARED((N_DEG,), jnp.float32),
    ],
)
def _sc_degree(dst_hbm, zeros_hbm, out_hbm, didx, ones_v, acc_sh):
    cid = lax.axis_index("c")
    sid = lax.axis_index("s")
    wid = cid * NS + sid
    for i in range(CHUNK // LANES):
        ones_v[pl.ds(i * LANES, LANES)] = jnp.ones((LANES,), jnp.float32)
    r0 = sid * ROWS_DEG
    pltpu.sync_copy(zeros_hbm.at[pl.ds(r0, ROWS_DEG)],
                    acc_sh.at[pl.ds(r0, ROWS_DEG)])
    pltpu.sync_copy(dst_hbm.at[pl.ds(wid * DEG_CH, DEG_CH)], didx)
    plsc.subcore_barrier()

    @pl.loop(0, DEG_CH)
    def _(j):
        pltpu.sync_copy(ones_v, acc_sh.at[didx.at[j]], add=True)

    plsc.subcore_barrier()
    pltpu.sync_copy(acc_sh.at[pl.ds(r0, ROWS_DEG)],
                    out_hbm.at[cid, pl.ds(r0, ROWS_DEG)])


def _make_sc_aggregate(h):
  @functools.partial(
    pl.kernel,
    out_type=jax.ShapeDtypeStruct((N_AGG, D_HID), jnp.float32),
    mesh=_mesh,
    scratch_types=[
        pltpu.VMEM((SLAB, CHUNK), jnp.int32),
        pltpu.VMEM((SLAB, CHUNK), jnp.int32),
        [pltpu.VMEM((CHUNK, D_HID), jnp.float32) for _ in range(NBUF)],
        pltpu.VMEM_SHARED((N_AGG, D_HID), jnp.float32),
        [pltpu.SemaphoreType.DMA for _ in range(NBUF)],
    ],
  )
  def _sc_aggregate(src_hbm, dst_hbm, table_hbm, zeros_hbm, out_hbm,
                  sidx, didx, rows, acc_sh, sems):
    cid = lax.axis_index("c")
    sid = lax.axis_index("s")
    r0 = sid * ROWS_AGG

    @pl.when(cid == 0)
    def _():
        c0 = sid * AGG_CH + h * (AGG_CH // 2)
        pltpu.sync_copy(zeros_hbm.at[pl.ds(r0, ROWS_AGG)],
                        acc_sh.at[pl.ds(r0, ROWS_AGG)])
        plsc.subcore_barrier()

        for sl in range(NSLAB // 2):
            pltpu.sync_copy(src_hbm.at[pl.ds(c0 + sl * SLAB, SLAB)], sidx)
            pltpu.sync_copy(dst_hbm.at[pl.ds(c0 + sl * SLAB, SLAB)], didx)
            for b in range(NBUF - 1):
                pltpu.async_copy(table_hbm.at[sidx.at[b]], rows[b], sems[b])

            @pl.loop(0, SLAB // NBUF)
            def _(q):
                for b in range(NBUF):
                    j = NBUF * q + b
                    pltpu.make_async_copy(table_hbm.at[sidx.at[j]], rows[b],
                                          sems[b]).wait()
                    bn = (b + NBUF - 1) % NBUF

                    @pl.when(j + NBUF - 1 < SLAB)
                    def _():
                        pltpu.async_copy(table_hbm.at[sidx.at[j + NBUF - 1]],
                                         rows[bn], sems[bn])

                    pltpu.sync_copy(rows[b], acc_sh.at[didx.at[j]], add=True)

        plsc.subcore_barrier()
        pltpu.sync_copy(acc_sh.at[pl.ds(r0, ROWS_AGG)],
                        out_hbm.at[pl.ds(r0, ROWS_AGG)])

  return _sc_aggregate


_sc_aggregate_a = _make_sc_aggregate(0)
_sc_aggregate_b = _make_sc_aggregate(1)


def _tc_pre(degp, x, W1):
    def body(degp_ref, x_ref, w_ref, dinv_ref, hs_ref):
        deg = degp_ref[0, :N, :] + degp_ref[1, :N, :] + 1.0
        dinv = lax.rsqrt(deg)
        h = jnp.dot(x_ref[...], w_ref[...], preferred_element_type=jnp.float32)
        dinv_ref[...] = dinv
        hs_ref[...] = h * dinv

    return pl.pallas_call(
        body,
        out_shape=(jax.ShapeDtypeStruct((N, 1), jnp.float32),
                   jax.ShapeDtypeStruct((N, D_HID), jnp.float32)),
    )(degp, x, W1)


def _tc_mid(agga, aggb, hs1, dinv, b1, gamma, beta):
    def body(agga_ref, aggb_ref, hs1_ref, dinv_ref, b1_ref, g_ref, be_ref,
             hs2_ref):
        dinv = dinv_ref[...]
        agg = agga_ref[:N, :] + aggb_ref[:N, :]
        h = dinv * (agg + hs1_ref[...]) + b1_ref[...]
        mean = jnp.mean(h, axis=0, keepdims=True)
        cen = h - mean
        var = jnp.mean(cen * cen, axis=0, keepdims=True)
        h = cen * lax.rsqrt(var + EPS) * g_ref[...] + be_ref[...]
        h = jnp.maximum(h, 0.0)
        hs2_ref[...] = h * dinv

    return pl.pallas_call(
        body,
        out_shape=jax.ShapeDtypeStruct((N, D_HID), jnp.float32),
    )(agga, aggb, hs1, dinv, b1, gamma, beta)


def _tc_post(agga, aggb, hs2, dinv, W2, b2):
    def body(agga_ref, aggb_ref, hs2_ref, dinv_ref, w2_ref, b2_ref, out_ref):
        t = dinv_ref[...] * (agga_ref[:N, :] + aggb_ref[:N, :] + hs2_ref[...])
        out_ref[...] = jnp.dot(t, w2_ref[...],
                               preferred_element_type=jnp.float32) + b2_ref[...]

    return pl.pallas_call(
        body,
        out_shape=jax.ShapeDtypeStruct((N, D_OUT), jnp.float32),
    )(agga, aggb, hs2, dinv, W2, b2)


def kernel(x, edge_index, W1, b1, gamma, beta, W2, b2):
    src = edge_index[0]
    dst = edge_index[1]
    pad = E_PAD - E
    srcp = jnp.concatenate([src, jnp.zeros((pad,), jnp.int32)])
    dstp = jnp.concatenate([dst, jnp.full((pad,), N, jnp.int32)])
    src2 = srcp.reshape(NROW, CHUNK)
    dst2 = dstp.reshape(NROW, CHUNK)

    zeros1 = jnp.zeros((N_DEG,), jnp.float32)
    degp = _sc_degree(dst2, zeros1)
    dinv, hs1 = _tc_pre(degp.reshape(NC, N_DEG, 1), x, W1)

    zeros_h = jnp.zeros((N_AGG, D_HID), jnp.float32)
    agg1a = _sc_aggregate_a(src2, dst2, hs1, zeros_h)
    agg1b = _sc_aggregate_b(src2, dst2, hs1, zeros_h)
    hs2 = _tc_mid(agg1a, agg1b, hs1, dinv,
                  b1.reshape(1, D_HID), gamma.reshape(1, D_HID),
                  beta.reshape(1, D_HID))

    agg2a = _sc_aggregate_a(src2, dst2, hs2, zeros_h)
    agg2b = _sc_aggregate_b(src2, dst2, hs2, zeros_h)
    out = _tc_post(agg2a, agg2b, hs2, dinv, W2, b2.reshape(1, D_OUT))
    return out


# speedup vs baseline: 1.1358x; 1.1358x over previous
import functools

import jax
import jax.numpy as jnp
from jax import lax
from jax.experimental import pallas as pl
from jax.experimental.pallas import tpu as pltpu
from jax.experimental.pallas import tpu_sc as plsc

N = 10000
E = 320000
D_IN = 128
D_HID = 128
D_OUT = 64
EPS = 1e-5

NC = 2
NS = 16
NW = NC * NS
LANES = 16
CHUNK = 128

NCH = 2 * ((E + 2 * NW * CHUNK - 1) // (2 * NW * CHUNK))
EW = NCH * CHUNK
E_PAD = EW * NW

ROWS_PER_TILE = 640
N_ACC = NS * ROWS_PER_TILE
ROWS_AGG = 632
N_AGG = NS * ROWS_AGG
HALF = NCH // 2

_mesh = plsc.VectorSubcoreMesh(core_axis_name="c", subcore_axis_name="s")


@functools.partial(
    pl.kernel,
    out_type=jax.ShapeDtypeStruct((NC, N_ACC), jnp.float32),
    mesh=_mesh,
    scratch_types=[
        pltpu.VMEM((NCH, CHUNK), jnp.int32),
        pltpu.VMEM((CHUNK,), jnp.float32),
        pltpu.VMEM_SHARED((N_ACC,), jnp.float32),
    ],
)
def _sc_degree(dst_hbm, zeros_hbm, out_hbm, didx, ones_v, acc_sh):
    cid = lax.axis_index("c")
    sid = lax.axis_index("s")
    wid = cid * NS + sid
    for i in range(CHUNK // LANES):
        ones_v[pl.ds(i * LANES, LANES)] = jnp.ones((LANES,), jnp.float32)
    r0 = sid * ROWS_PER_TILE
    pltpu.sync_copy(zeros_hbm.at[pl.ds(r0, ROWS_PER_TILE)],
                    acc_sh.at[pl.ds(r0, ROWS_PER_TILE)])
    pltpu.sync_copy(dst_hbm.at[pl.ds(wid * NCH, NCH)], didx)
    plsc.subcore_barrier()

    @pl.loop(0, NCH)
    def _(j):
        pltpu.sync_copy(ones_v, acc_sh.at[didx.at[j]], add=True)

    plsc.subcore_barrier()
    pltpu.sync_copy(acc_sh.at[pl.ds(r0, ROWS_PER_TILE)],
                    out_hbm.at[cid, pl.ds(r0, ROWS_PER_TILE)])


def _make_sc_aggregate(D):
    @functools.partial(
        pl.kernel,
        out_type=jax.ShapeDtypeStruct((NC, N_AGG, D), jnp.float32),
        mesh=_mesh,
        scratch_types=[
            pltpu.VMEM((HALF, CHUNK), jnp.int32),
            pltpu.VMEM((HALF, CHUNK), jnp.int32),
            pltpu.VMEM((CHUNK, D), jnp.float32),
            pltpu.VMEM((CHUNK, D), jnp.float32),
            pltpu.VMEM_SHARED((N_AGG, D), jnp.float32),
            pltpu.SemaphoreType.DMA,
            pltpu.SemaphoreType.DMA,
        ],
    )
    def _sc_aggregate(src_hbm, dst_hbm, table_hbm, zeros_hbm, out_hbm,
                      sidx, didx, rows0, rows1, acc_sh, sem0, sem1):
        cid = lax.axis_index("c")
        sid = lax.axis_index("s")
        wid = cid * NS + sid
        r0 = sid * ROWS_AGG
        pltpu.sync_copy(zeros_hbm.at[pl.ds(r0, ROWS_AGG)],
                        acc_sh.at[pl.ds(r0, ROWS_AGG)])
        c0 = wid * NCH
        plsc.subcore_barrier()

        for ph in range(2):
            pltpu.sync_copy(src_hbm.at[pl.ds(c0 + ph * HALF, HALF)], sidx)
            pltpu.sync_copy(dst_hbm.at[pl.ds(c0 + ph * HALF, HALF)], didx)
            @pl.loop(0, HALF)
            def _(j):
                pltpu.async_copy(table_hbm.at[sidx.at[j]], rows0, sem0).wait()
                pltpu.sync_copy(rows0, acc_sh.at[didx.at[j]], add=True)

        plsc.subcore_barrier()
        pltpu.sync_copy(acc_sh.at[pl.ds(r0, ROWS_AGG)],
                        out_hbm.at[cid, pl.ds(r0, ROWS_AGG)])

    return _sc_aggregate


_sc_aggregate_hid = _make_sc_aggregate(D_HID)


def _tc_pre(degp, x, W1):
    def body(degp_ref, x_ref, w_ref, dinv_ref, hs_ref):
        deg = degp_ref[0, :N, :] + degp_ref[1, :N, :] + 1.0
        dinv = lax.rsqrt(deg)
        h = jnp.dot(x_ref[...], w_ref[...], preferred_element_type=jnp.float32)
        dinv_ref[...] = dinv
        hs_ref[...] = h * dinv

    return pl.pallas_call(
        body,
        out_shape=(jax.ShapeDtypeStruct((N, 1), jnp.float32),
                   jax.ShapeDtypeStruct((N, D_HID), jnp.float32)),
    )(degp, x, W1)


def _tc_mid(aggp, hs1, dinv, b1, gamma, beta):
    def body(aggp_ref, hs1_ref, dinv_ref, b1_ref, g_ref, be_ref, hs2_ref):
        agg = aggp_ref[0, :N, :] + aggp_ref[1, :N, :]
        dinv = dinv_ref[...]
        h = dinv * (agg + hs1_ref[...]) + b1_ref[...]
        mean = jnp.mean(h, axis=0, keepdims=True)
        cen = h - mean
        var = jnp.mean(cen * cen, axis=0, keepdims=True)
        h = cen * lax.rsqrt(var + EPS) * g_ref[...] + be_ref[...]
        h = jnp.maximum(h, 0.0)
        hs2_ref[...] = h * dinv

    return pl.pallas_call(
        body,
        out_shape=jax.ShapeDtypeStruct((N, D_HID), jnp.float32),
    )(aggp, hs1, dinv, b1, gamma, beta)


def _tc_post(aggp, hs2, dinv, W2, b2):
    def body(aggp_ref, hs2_ref, dinv_ref, w2_ref, b2_ref, out_ref):
        agg = aggp_ref[0, :N, :] + aggp_ref[1, :N, :]
        t = dinv_ref[...] * (agg + hs2_ref[...])
        out_ref[...] = jnp.dot(t, w2_ref[...],
                               preferred_element_type=jnp.float32) + b2_ref[...]

    return pl.pallas_call(
        body,
        out_shape=jax.ShapeDtypeStruct((N, D_OUT), jnp.float32),
    )(aggp, hs2, dinv, W2, b2)


def kernel(x, edge_index, W1, b1, gamma, beta, W2, b2):
    src = edge_index[0]
    dst = edge_index[1]
    pad = E_PAD - E
    srcp = jnp.concatenate([src, jnp.zeros((pad,), jnp.int32)])
    dstp = jnp.concatenate([dst, jnp.full((pad,), N, jnp.int32)])
    src2 = srcp.reshape(NW * NCH, CHUNK)
    dst2 = dstp.reshape(NW * NCH, CHUNK)

    zeros1 = jnp.zeros((N_ACC,), jnp.float32)
    degp = _sc_degree(dst2, zeros1)
    dinv, hs1 = _tc_pre(degp.reshape(NC, N_ACC, 1), x, W1)

    zeros_h = jnp.zeros((N_AGG, D_HID), jnp.float32)
    aggp1 = _sc_aggregate_hid(src2, dst2, hs1, zeros_h)
    hs2 = _tc_mid(aggp1, hs1, dinv,
                  b1.reshape(1, D_HID), gamma.reshape(1, D_HID),
                  beta.reshape(1, D_HID))

    aggp2 = _sc_aggregate_hid(src2, dst2, hs2, zeros_h)
    out = _tc_post(aggp2, hs2, dinv, W2, b2.reshape(1, D_OUT))
    return out


# speedup vs baseline: 1.4723x; 1.2963x over previous
import functools

import jax
import jax.numpy as jnp
from jax import lax
from jax.experimental import pallas as pl
from jax.experimental.pallas import tpu as pltpu
from jax.experimental.pallas import tpu_sc as plsc

N = 10000
E = 320000
D_IN = 128
D_HID = 128
D_OUT = 64
EPS = 1e-5

NC = 2
NS = 16
NW = NC * NS
LANES = 16
CHUNK = 128

EW = ((E + NW * CHUNK - 1) // (NW * CHUNK)) * CHUNK
E_PAD = EW * NW
NCH = EW // CHUNK

ROWS_PER_TILE = 640
N_ACC = NS * ROWS_PER_TILE

_mesh = plsc.VectorSubcoreMesh(core_axis_name="c", subcore_axis_name="s")


@functools.partial(
    pl.kernel,
    out_type=jax.ShapeDtypeStruct((NC, N_ACC), jnp.float32),
    mesh=_mesh,
    scratch_types=[
        pltpu.VMEM((CHUNK,), jnp.int32),
        pltpu.VMEM((CHUNK,), jnp.float32),
        pltpu.VMEM_SHARED((N_ACC,), jnp.float32),
    ],
)
def _sc_degree(dst_hbm, zeros_hbm, out_hbm, idx_v, ones_v, acc_sh):
    cid = lax.axis_index("c")
    sid = lax.axis_index("s")
    wid = cid * NS + sid
    for i in range(CHUNK // LANES):
        ones_v[pl.ds(i * LANES, LANES)] = jnp.ones((LANES,), jnp.float32)
    r0 = sid * ROWS_PER_TILE
    pltpu.sync_copy(zeros_hbm.at[pl.ds(r0, ROWS_PER_TILE)],
                    acc_sh.at[pl.ds(r0, ROWS_PER_TILE)])
    plsc.subcore_barrier()
    base0 = wid * EW

    @pl.loop(0, NCH)
    def _(j):
        base = base0 + j * CHUNK
        pltpu.sync_copy(dst_hbm.at[pl.ds(base, CHUNK)], idx_v)
        pltpu.sync_copy(ones_v, acc_sh.at[idx_v], add=True)

    plsc.subcore_barrier()
    pltpu.sync_copy(acc_sh.at[pl.ds(r0, ROWS_PER_TILE)],
                    out_hbm.at[cid, pl.ds(r0, ROWS_PER_TILE)])


def _make_sc_aggregate(D):
    @functools.partial(
        pl.kernel,
        out_type=jax.ShapeDtypeStruct((NC, N_ACC, D), jnp.float32),
        mesh=_mesh,
        scratch_types=[
            pltpu.VMEM((CHUNK,), jnp.int32),
            pltpu.VMEM((CHUNK,), jnp.int32),
            pltpu.VMEM((CHUNK, D), jnp.float32),
            pltpu.VMEM_SHARED((N_ACC, D), jnp.float32),
            pltpu.SemaphoreType.DMA,
        ],
    )
    def _sc_aggregate(src_hbm, dst_hbm, table_hbm, zeros_hbm, out_hbm,
                      sidx, didx, rows, acc_sh, sem):
        cid = lax.axis_index("c")
        sid = lax.axis_index("s")
        wid = cid * NS + sid
        r0 = sid * ROWS_PER_TILE
        pltpu.sync_copy(zeros_hbm.at[pl.ds(r0, ROWS_PER_TILE)],
                        acc_sh.at[pl.ds(r0, ROWS_PER_TILE)])
        plsc.subcore_barrier()
        base0 = wid * EW

        @pl.loop(0, NCH)
        def _(j):
            base = base0 + j * CHUNK
            pltpu.sync_copy(src_hbm.at[pl.ds(base, CHUNK)], sidx)
            cp = pltpu.async_copy(table_hbm.at[sidx], rows, sem)
            pltpu.sync_copy(dst_hbm.at[pl.ds(base, CHUNK)], didx)
            cp.wait()
            pltpu.sync_copy(rows, acc_sh.at[didx], add=True)

        plsc.subcore_barrier()
        pltpu.sync_copy(acc_sh.at[pl.ds(r0, ROWS_PER_TILE)],
                        out_hbm.at[cid, pl.ds(r0, ROWS_PER_TILE)])

    return _sc_aggregate


_sc_aggregate_hid = _make_sc_aggregate(D_HID)


def _tc_pre(degp, x, W1):
    def body(degp_ref, x_ref, w_ref, dinv_ref, hs_ref):
        deg = degp_ref[0, :N, :] + degp_ref[1, :N, :] + 1.0
        dinv = lax.rsqrt(deg)
        h = jnp.dot(x_ref[...], w_ref[...], preferred_element_type=jnp.float32)
        dinv_ref[...] = dinv
        hs_ref[...] = h * dinv

    return pl.pallas_call(
        body,
        out_shape=(jax.ShapeDtypeStruct((N, 1), jnp.float32),
                   jax.ShapeDtypeStruct((N, D_HID), jnp.float32)),
    )(degp, x, W1)


def _tc_mid(aggp, hs1, dinv, b1, gamma, beta):
    def body(aggp_ref, hs1_ref, dinv_ref, b1_ref, g_ref, be_ref, hs2_ref):
        agg = aggp_ref[0, :N, :] + aggp_ref[1, :N, :]
        dinv = dinv_ref[...]
        h = dinv * (agg + hs1_ref[...]) + b1_ref[...]
        mean = jnp.mean(h, axis=0, keepdims=True)
        cen = h - mean
        var = jnp.mean(cen * cen, axis=0, keepdims=True)
        h = cen * lax.rsqrt(var + EPS) * g_ref[...] + be_ref[...]
        h = jnp.maximum(h, 0.0)
        hs2_ref[...] = h * dinv

    return pl.pallas_call(
        body,
        out_shape=jax.ShapeDtypeStruct((N, D_HID), jnp.float32),
    )(aggp, hs1, dinv, b1, gamma, beta)


def _tc_post(aggp, hs2, dinv, W2, b2):
    def body(aggp_ref, hs2_ref, dinv_ref, w2_ref, b2_ref, out_ref):
        agg = aggp_ref[0, :N, :] + aggp_ref[1, :N, :]
        t = dinv_ref[...] * (agg + hs2_ref[...])
        out_ref[...] = jnp.dot(t, w2_ref[...],
                               preferred_element_type=jnp.float32) + b2_ref[...]

    return pl.pallas_call(
        body,
        out_shape=jax.ShapeDtypeStruct((N, D_OUT), jnp.float32),
    )(aggp, hs2, dinv, W2, b2)


def kernel(x, edge_index, W1, b1, gamma, beta, W2, b2):
    src = edge_index[0]
    dst = edge_index[1]
    pad = E_PAD - E
    srcp = jnp.concatenate([src, jnp.zeros((pad,), jnp.int32)])
    dstp = jnp.concatenate([dst, jnp.full((pad,), N, jnp.int32)])

    zeros1 = jnp.zeros((N_ACC,), jnp.float32)
    degp = _sc_degree(dstp, zeros1)
    dinv, hs1 = _tc_pre(degp.reshape(NC, N_ACC, 1), x, W1)

    zeros_h = jnp.zeros((N_ACC, D_HID), jnp.float32)
    aggp1 = _sc_aggregate_hid(srcp, dstp, hs1, zeros_h)
    hs2 = _tc_mid(aggp1, hs1, dinv,
                  b1.reshape(1, D_HID), gamma.reshape(1, D_HID),
                  beta.reshape(1, D_HID))

    aggp2 = _sc_aggregate_hid(srcp, dstp, hs2, zeros_h)
    out = _tc_post(aggp2, hs2, dinv, W2, b2.reshape(1, D_OUT))
    return out
